# baseline (speedup 1.0000x reference)
import functools

import jax
import jax.numpy as jnp
from jax import lax
from jax.experimental import pallas as pl
from jax.experimental.pallas import tpu as pltpu
from jax.experimental.pallas import tpu_sc as plsc

B, P, C = 16384, 26, 128
TOTAL = B * P
NC, NS, L = 2, 16, 16
NW = NC * NS
ROWS_PER_W = TOTAL // NW
R = 256
CHUNKS = ROWS_PER_W // R
BUF = R * C


def _body(idx_hbm, out_hbm, idx_v, rows0, rows1, sem_i, sem0, sem1):
    wid = lax.axis_index("s") * NC + lax.axis_index("c")
    base_row = wid * ROWS_PER_W

    idx_cp = pltpu.async_copy(
        idx_hbm.at[pl.ds(base_row * 1, ROWS_PER_W)], idx_v, sem_i)

    zeros = jnp.zeros((L,), jnp.float32)
    ones = jnp.ones((L,), jnp.float32)
    lanes = lax.iota(jnp.int32, L)

    def zero_buf(i):
        for k in range(C // L):
            rows0[i, pl.ds(k * L, L)] = zeros
            rows1[i, pl.ds(k * L, L)] = zeros

    pl.loop(0, R)(zero_buf)
    idx_cp.wait()

    bufs = (rows0, rows1)
    sems = (sem0, sem1)

    def scatter(chunk, buf, val):
        for j in range(R // L):
            iv = idx_v[pl.ds(chunk * R + j * L, L)]
            plsc.store_scatter(buf, [lanes + j * L, iv], val)

    def fire(chunk, buf, sem):
        dst = out_hbm.at[pl.ds(base_row + chunk * R, R)]
        pltpu.async_copy(buf, dst, sem)

    def drain(buf, sem):
        dst = out_hbm.at[pl.ds(0, R)]
        pltpu.make_async_copy(buf, dst, sem).wait()

    for b in range(2):
        scatter(b, bufs[b], ones)
        fire(b, bufs[b], sems[b])

    def step(c):
        for b in range(2):
            chunk = c + b
            drain(bufs[b], sems[b])
            scatter(chunk - 2, bufs[b], zeros)
            scatter(chunk, bufs[b], ones)
            fire(chunk, bufs[b], sems[b])

    pl.loop(2, CHUNKS, step=2)(step)

    for b in range(2):
        drain(bufs[b], sems[b])


@functools.partial(jax.jit, static_argnames=())
def kernel(index_feature):
    idx_flat = index_feature.reshape(TOTAL).astype(jnp.int32)
    sc_kernel = pl.kernel(
        _body,
        out_type=jax.ShapeDtypeStruct((TOTAL, C), jnp.float32),
        mesh=plsc.VectorSubcoreMesh(
            core_axis_name="c", subcore_axis_name="s",
            num_cores=NC, num_subcores=NS),
        compiler_params=pltpu.CompilerParams(needs_layout_passes=False),
        scratch_types=[
            pltpu.VMEM((ROWS_PER_W,), jnp.int32),
            pltpu.VMEM((R, C), jnp.float32),
            pltpu.VMEM((R, C), jnp.float32),
            pltpu.SemaphoreType.DMA,
            pltpu.SemaphoreType.DMA,
            pltpu.SemaphoreType.DMA,
        ],
    )
    out = sc_kernel(idx_flat)
    return out.reshape(B, P, C)


# speedup vs baseline: 1.8726x; 1.8726x over previous
import functools

import jax
import jax.numpy as jnp
from jax import lax
from jax.experimental import pallas as pl
from jax.experimental.pallas import tpu as pltpu
from jax.experimental.pallas import tpu_sc as plsc

B, P, C = 16384, 26, 128
TOTAL = B * P
NC, NS, L = 2, 16, 16
NW = NC * NS
BATCH_PER_W = B // NW
ROWS_PER_W = BATCH_PER_W * P
NB = 8
RPC = NB * P
CHUNKS = BATCH_PER_W // NB


def _body(idx_hbm, out_hbm, idx_v, rows0, rows1, sem_i, sem0, sem1):
    wid = lax.axis_index("s") * NC + lax.axis_index("c")
    base_b = wid * BATCH_PER_W

    idx_cp = pltpu.async_copy(
        idx_hbm.at[pl.ds(wid * ROWS_PER_W, ROWS_PER_W)], idx_v, sem_i)

    zeros = jnp.zeros((L,), jnp.float32)
    ones = jnp.ones((L,), jnp.float32)
    lanes = lax.iota(jnp.int32, L)

    def zero_buf(i):
        for k in range(C // L):
            rows0[i, pl.ds(k * L, L)] = zeros
            rows1[i, pl.ds(k * L, L)] = zeros

    pl.loop(0, RPC)(zero_buf)
    idx_cp.wait()

    bufs = (rows0, rows1)
    sems = (sem0, sem1)

    def scatter(chunk, buf, val):
        for j in range(RPC // L):
            iv = idx_v[pl.ds(chunk * RPC + j * L, L)]
            plsc.store_scatter(buf, [lanes + j * L, iv], val)

    def fire(chunk, buf, sem):
        for b in range(NB):
            dst = out_hbm.at[base_b + chunk * NB + b]
            pltpu.async_copy(buf.at[pl.ds(b * P, P)], dst, sem)

    def drain(buf, sem):
        for b in range(NB):
            pltpu.make_async_copy(
                buf.at[pl.ds(0, P)], out_hbm.at[0], sem).wait()

    for b in range(2):
        scatter(b, bufs[b], ones)
        fire(b, bufs[b], sems[b])

    def step(c):
        for b in range(2):
            chunk = c + b
            drain(bufs[b], sems[b])
            scatter(chunk - 2, bufs[b], zeros)
            scatter(chunk, bufs[b], ones)
            fire(chunk, bufs[b], sems[b])

    pl.loop(2, CHUNKS, step=2)(step)

    for b in range(2):
        drain(bufs[b], sems[b])


@functools.partial(jax.jit, static_argnames=())
def kernel(index_feature):
    idx_flat = index_feature.reshape(TOTAL).astype(jnp.int32)
    sc_kernel = pl.kernel(
        _body,
        out_type=jax.ShapeDtypeStruct((B, P, C), jnp.float32),
        mesh=plsc.VectorSubcoreMesh(
            core_axis_name="c", subcore_axis_name="s",
            num_cores=NC, num_subcores=NS),
        compiler_params=pltpu.CompilerParams(needs_layout_passes=False),
        scratch_types=[
            pltpu.VMEM((ROWS_PER_W,), jnp.int32),
            pltpu.VMEM((RPC, C), jnp.float32),
            pltpu.VMEM((RPC, C), jnp.float32),
            pltpu.SemaphoreType.DMA,
            pltpu.SemaphoreType.DMA,
            pltpu.SemaphoreType.DMA,
        ],
    )
    return sc_kernel(idx_flat)


# speedup vs baseline: 4.9749x; 2.6567x over previous
import functools

import jax
import jax.numpy as jnp
from jax import lax
from jax.experimental import pallas as pl
from jax.experimental.pallas import tpu as pltpu
from jax.experimental.pallas import tpu_sc as plsc

B, P, C = 16384, 26, 128
TOTAL = B * P
NC, NS, L = 2, 16, 16
NW = NC * NS
BATCH_PER_W = B // NW
ROWS_PER_W = BATCH_PER_W * P
NB = 16
RPC = NB * P
CHUNKS = BATCH_PER_W // NB


def _body(idx_hbm, out_hbm, idx_v, rows0, rows1, sem_i, sem0, sem1):
    wid = lax.axis_index("s") * NC + lax.axis_index("c")
    base_b = wid * BATCH_PER_W

    idx_cp = pltpu.async_copy(
        idx_hbm.at[pl.ds(wid * ROWS_PER_W, ROWS_PER_W)], idx_v, sem_i)

    zeros = jnp.zeros((L,), jnp.float32)
    ones = jnp.ones((L,), jnp.float32)
    lanes = lax.iota(jnp.int32, L)
    trows = []
    for j in range(RPC // L):
        t = lanes + (j * L)
        bi = t // P
        trows.append((t - bi * P) * NB + bi)

    def zero_buf(i):
        for k in range(C // L):
            rows0[i, pl.ds(k * L, L)] = zeros
            rows1[i, pl.ds(k * L, L)] = zeros

    pl.loop(0, RPC)(zero_buf)
    idx_cp.wait()

    bufs = (rows0, rows1)
    sems = (sem0, sem1)

    def scatter(chunk, buf, val):
        for j in range(RPC // L):
            iv = idx_v[pl.ds(chunk * RPC + j * L, L)]
            plsc.store_scatter(buf, [trows[j], iv], val)

    def fire(chunk, buf, sem):
        for p in range(P):
            dst = out_hbm.at[p, pl.ds(base_b + chunk * NB, NB)]
            pltpu.async_copy(buf.at[pl.ds(p * NB, NB)], dst, sem)

    def drain(buf, sem):
        for p in range(P):
            pltpu.make_async_copy(
                buf.at[pl.ds(0, NB)], out_hbm.at[0, pl.ds(0, NB)],
                sem).wait()

    for b in range(2):
        scatter(b, bufs[b], ones)
        fire(b, bufs[b], sems[b])

    def step(c):
        for b in range(2):
            chunk = c + b
            drain(bufs[b], sems[b])
            scatter(chunk - 2, bufs[b], zeros)
            scatter(chunk, bufs[b], ones)
            fire(chunk, bufs[b], sems[b])

    pl.loop(2, CHUNKS, step=2)(step)

    for b in range(2):
        drain(bufs[b], sems[b])


@functools.partial(jax.jit, static_argnames=())
def kernel(index_feature):
    idx_flat = index_feature.reshape(TOTAL).astype(jnp.int32)
    sc_kernel = pl.kernel(
        _body,
        out_type=jax.ShapeDtypeStruct((P, B, C), jnp.float32),
        mesh=plsc.VectorSubcoreMesh(
            core_axis_name="c", subcore_axis_name="s",
            num_cores=NC, num_subcores=NS),
        compiler_params=pltpu.CompilerParams(needs_layout_passes=False),
        scratch_types=[
            pltpu.VMEM((ROWS_PER_W,), jnp.int32),
            pltpu.VMEM((RPC, C), jnp.float32),
            pltpu.VMEM((RPC, C), jnp.float32),
            pltpu.SemaphoreType.DMA,
            pltpu.SemaphoreType.DMA,
            pltpu.SemaphoreType.DMA,
        ],
    )
    out_pm = sc_kernel(idx_flat)
    return jnp.transpose(out_pm, (1, 0, 2))


# speedup vs baseline: 5.5831x; 1.1223x over previous
import functools

import jax
import jax.numpy as jnp
from jax import lax
from jax.experimental import pallas as pl
from jax.experimental.pallas import tpu as pltpu
from jax.experimental.pallas import tpu_sc as plsc

B, P, C = 16384, 26, 128
NC, NS, L = 2, 16, 16
NW = NC * NS
BATCH_PER_W = B // NW
NB = 16
RPC = NB * P
CHUNKS = BATCH_PER_W // NB


def _body(idx_hbm, out_hbm, idx_v, rows0, rows1, sem_i, sem0, sem1):
    wid = lax.axis_index("s") * NC + lax.axis_index("c")
    base_b = wid * BATCH_PER_W

    for p in range(P):
        pltpu.async_copy(
            idx_hbm.at[p, pl.ds(base_b, BATCH_PER_W)], idx_v.at[p], sem_i)

    zeros = jnp.zeros((L,), jnp.float32)
    ones = jnp.ones((L,), jnp.float32)
    lanes = lax.iota(jnp.int32, L)

    bufs = (rows0, rows1)
    sems = (sem0, sem1)

    def zero_buf(buf):
        def zrow(i):
            for k in range(C // L):
                buf[i, pl.ds(k * L, L)] = zeros
        pl.loop(0, RPC)(zrow)

    def scatter(chunk, buf, val):
        for p in range(P):
            iv = idx_v[p, pl.ds(chunk * NB, NB)]
            plsc.store_scatter(buf, [lanes + p * NB, iv], val)

    def fire(chunk, buf, sem):
        for p in range(P):
            dst = out_hbm.at[p, pl.ds(base_b + chunk * NB, NB)]
            pltpu.async_copy(buf.at[pl.ds(p * NB, NB)], dst, sem)

    def drain(buf, sem):
        for p in range(P):
            pltpu.make_async_copy(
                buf.at[pl.ds(0, NB)], out_hbm.at[0, pl.ds(0, NB)],
                sem).wait()

    def wait_idx():
        for p in range(P):
            pltpu.make_async_copy(
                idx_hbm.at[0, pl.ds(0, BATCH_PER_W)], idx_v.at[0],
                sem_i).wait()

    zero_buf(rows0)
    wait_idx()
    scatter(0, rows0, ones)
    fire(0, rows0, sem0)
    zero_buf(rows1)
    scatter(1, rows1, ones)
    fire(1, rows1, sem1)

    def step(c):
        for b in range(2):
            chunk = c + b
            drain(bufs[b], sems[b])
            scatter(chunk - 2, bufs[b], zeros)
            scatter(chunk, bufs[b], ones)
            fire(chunk, bufs[b], sems[b])

    pl.loop(2, CHUNKS, step=2)(step)

    for b in range(2):
        drain(bufs[b], sems[b])


@functools.partial(jax.jit, static_argnames=())
def kernel(index_feature):
    idx_t = jnp.transpose(index_feature).astype(jnp.int32)
    sc_kernel = pl.kernel(
        _body,
        out_type=jax.ShapeDtypeStruct((P, B, C), jnp.float32),
        mesh=plsc.VectorSubcoreMesh(
            core_axis_name="c", subcore_axis_name="s",
            num_cores=NC, num_subcores=NS),
        compiler_params=pltpu.CompilerParams(needs_layout_passes=False),
        scratch_types=[
            pltpu.VMEM((P, BATCH_PER_W), jnp.int32),
            pltpu.VMEM((RPC, C), jnp.float32),
            pltpu.VMEM((RPC, C), jnp.float32),
            pltpu.SemaphoreType.DMA,
            pltpu.SemaphoreType.DMA,
            pltpu.SemaphoreType.DMA,
        ],
    )
    out_pm = sc_kernel(idx_t)
    return jnp.transpose(out_pm, (1, 0, 2))


# speedup vs baseline: 5.7703x; 1.0335x over previous
import functools

import jax
import jax.numpy as jnp
from jax import lax
from jax.experimental import pallas as pl
from jax.experimental.pallas import tpu as pltpu
from jax.experimental.pallas import tpu_sc as plsc

B, P, C = 16384, 26, 128
NC, NS, L = 2, 16, 16
NW = NC * NS
BATCH_PER_W = B // NW
NB = 16
RPC = NB * P
CHUNKS = BATCH_PER_W // NB


def _body(idx_hbm, out_hbm, idx_v, rows0, rows1, sem_i, sem0, sem1):
    wid = lax.axis_index("s") * NC + lax.axis_index("c")
    base_b = wid * BATCH_PER_W

    def stage_idx(p):
        pltpu.async_copy(
            idx_hbm.at[p, pl.ds(base_b, BATCH_PER_W)], idx_v.at[p], sem_i)

    pl.loop(0, P)(stage_idx)

    zeros = jnp.zeros((L,), jnp.float32)
    ones = jnp.ones((L,), jnp.float32)
    lanes = lax.iota(jnp.int32, L)

    bufs = (rows0, rows1)
    sems = (sem0, sem1)

    def zero_buf(buf):
        def zrow(i):
            for k in range(C // L):
                buf[i, pl.ds(k * L, L)] = zeros
        pl.loop(0, RPC)(zrow)

    def scatter(chunk, buf, val):
        def srow(p):
            iv = idx_v[p, pl.ds(chunk * NB, NB)]
            plsc.store_scatter(buf, [lanes + p * NB, iv], val)
        pl.loop(0, P)(srow)

    def fire(chunk, buf, sem):
        def frow(p):
            dst = out_hbm.at[p, pl.ds(base_b + chunk * NB, NB)]
            pltpu.async_copy(buf.at[pl.ds(p * NB, NB)], dst, sem)
        pl.loop(0, P)(frow)

    def drain(buf, sem):
        def drow(p):
            pltpu.make_async_copy(
                buf.at[pl.ds(0, NB)], out_hbm.at[0, pl.ds(0, NB)],
                sem).wait()
        pl.loop(0, P)(drow)

    def wait_idx():
        def wrow(p):
            pltpu.make_async_copy(
                idx_hbm.at[0, pl.ds(0, BATCH_PER_W)], idx_v.at[0],
                sem_i).wait()
        pl.loop(0, P)(wrow)

    zero_buf(rows0)
    wait_idx()
    scatter(0, rows0, ones)
    fire(0, rows0, sem0)
    zero_buf(rows1)
    scatter(1, rows1, ones)
    fire(1, rows1, sem1)

    def step(c):
        for b in range(2):
            chunk = c + b
            drain(bufs[b], sems[b])
            scatter(chunk - 2, bufs[b], zeros)
            scatter(chunk, bufs[b], ones)
            fire(chunk, bufs[b], sems[b])

    pl.loop(2, CHUNKS, step=2)(step)

    for b in range(2):
        drain(bufs[b], sems[b])


@functools.partial(jax.jit, static_argnames=())
def kernel(index_feature):
    idx_t = jnp.transpose(index_feature).astype(jnp.int32)
    sc_kernel = pl.kernel(
        _body,
        out_type=jax.ShapeDtypeStruct((P, B, C), jnp.float32),
        mesh=plsc.VectorSubcoreMesh(
            core_axis_name="c", subcore_axis_name="s",
            num_cores=NC, num_subcores=NS),
        compiler_params=pltpu.CompilerParams(needs_layout_passes=False),
        scratch_types=[
            pltpu.VMEM((P, BATCH_PER_W), jnp.int32),
            pltpu.VMEM((RPC, C), jnp.float32),
            pltpu.VMEM((RPC, C), jnp.float32),
            pltpu.SemaphoreType.DMA,
            pltpu.SemaphoreType.DMA,
            pltpu.SemaphoreType.DMA,
        ],
    )
    out_pm = sc_kernel(idx_t)
    return jnp.transpose(out_pm, (1, 0, 2))
